# Initial kernel scaffold; baseline (speedup 1.0000x reference)
import jax
import jax.numpy as jnp
from jax.experimental import pallas as pl


def kernel(h_drug, d_disease, edge_index, W, b, W1, b1, g1, be1, W2, b2, g2, be2, W3, b3):
    raise NotImplementedError("write your pallas kernel here")


# speedup vs baseline: 11.0935x; 11.0935x over previous
import functools

import jax
import jax.numpy as jnp
from jax import lax
from jax.experimental import pallas as pl
from jax.experimental.pallas import tpu as pltpu
from jax.experimental.pallas import tpu_sc as plsc

NC = 2
NS = 16
NW = NC * NS
K = 80


def _mesh():
    return plsc.VectorSubcoreMesh(core_axis_name="c", subcore_axis_name="s")


def _make_sc_degrees(NW_, NCH, K_, NB, w):
    rows_per_tile = NB // NS

    @functools.partial(
        pl.kernel,
        mesh=_mesh(),
        out_type=jax.ShapeDtypeStruct((NC * NB, w), jnp.float32),
        scratch_types=[
            pltpu.VMEM((NCH, K_), jnp.int32),
            pltpu.VMEM((NCH, K_), jnp.int32),
            pltpu.VMEM((K_, w), jnp.float32),
            pltpu.VMEM((K_, w), jnp.float32),
            pltpu.VMEM_SHARED((NB, w), jnp.float32),
        ],
    )
    def sc_degrees(src3, dst3, onesA, zerosW, degp, sidx_v, didx_v,
                   ones10_v, ones01_v, hist):
        cid = lax.axis_index("c")
        sid = lax.axis_index("s")
        wid = cid * NS + sid
        pltpu.sync_copy(src3.at[wid], sidx_v)
        pltpu.sync_copy(dst3.at[wid], didx_v)
        pltpu.sync_copy(onesA.at[0], ones10_v)
        pltpu.sync_copy(onesA.at[1], ones01_v)
        pltpu.sync_copy(zerosW.at[pl.ds(sid * rows_per_tile, rows_per_tile)],
                        hist.at[pl.ds(sid * rows_per_tile, rows_per_tile)])
        plsc.subcore_barrier()

        def body(j, carry):
            pltpu.sync_copy(ones10_v, hist.at[sidx_v.at[j]], add=True)
            pltpu.sync_copy(ones01_v, hist.at[didx_v.at[j]], add=True)
            return carry

        lax.fori_loop(0, NCH, body, 0)
        plsc.subcore_barrier()
        pltpu.sync_copy(
            hist.at[pl.ds(sid * rows_per_tile, rows_per_tile)],
            degp.at[pl.ds(cid * NB + sid * rows_per_tile, rows_per_tile)])

    return sc_degrees


def _make_sc_agg(NW_, NCH, K_, NB, dim, n_rows):
    rows_per_tile = NB // NS

    @functools.partial(
        pl.kernel,
        mesh=_mesh(),
        out_type=jax.ShapeDtypeStruct((NC * NB, dim), jnp.float32),
        scratch_types=[
            pltpu.VMEM((NCH, K_), jnp.int32),
            pltpu.VMEM((NCH, K_), jnp.int32),
            pltpu.VMEM((K_, dim), jnp.float32),
            pltpu.VMEM_SHARED((NB, dim), jnp.float32),
            pltpu.SemaphoreType.DMA,
        ],
    )
    def sc_agg(hs, src3, dst3, zerosD, aggp, sidx_v, didx_v, rows_v, agg, sem):
        cid = lax.axis_index("c")
        sid = lax.axis_index("s")
        wid = cid * NS + sid
        pltpu.sync_copy(src3.at[wid], sidx_v)
        pltpu.sync_copy(dst3.at[wid], didx_v)
        pltpu.sync_copy(zerosD.at[pl.ds(sid * rows_per_tile, rows_per_tile)],
                        agg.at[pl.ds(sid * rows_per_tile, rows_per_tile)])
        plsc.subcore_barrier()

        def body(j, carry):
            pltpu.async_copy(hs.at[sidx_v.at[j]], rows_v, sem).wait()
            pltpu.sync_copy(rows_v, agg.at[didx_v.at[j]], add=True)
            return carry

        lax.fori_loop(0, NCH, body, 0)
        plsc.subcore_barrier()
        pltpu.sync_copy(
            agg.at[pl.ds(sid * rows_per_tile, rows_per_tile)],
            aggp.at[pl.ds(cid * NB + sid * rows_per_tile, rows_per_tile)])

    return sc_agg


def _make_sc_gather(NW_, NCH, K_, NB):
    @functools.partial(
        pl.kernel,
        mesh=_mesh(),
        out_type=jax.ShapeDtypeStruct((NW, NCH, K_), jnp.float32),
        scratch_types=[
            pltpu.VMEM((NCH, K_), jnp.int32),
            pltpu.VMEM((NCH, K_), jnp.float32),
            pltpu.SemaphoreType.DMA,
        ],
    )
    def sc_gather(s1d, dst3, score3, didx_v, out_v, sem):
        cid = lax.axis_index("c")
        sid = lax.axis_index("s")
        wid = cid * NS + sid
        pltpu.sync_copy(dst3.at[wid], didx_v)

        def body(j, carry):
            pltpu.async_copy(s1d.at[didx_v.at[j]], out_v.at[j], sem).wait()
            return carry

        lax.fori_loop(0, NCH, body, 0)
        pltpu.sync_copy(out_v, score3.at[wid])

    return sc_gather


def _tc_scale_body(degp_ref, h_ref, hs_ref, *, NB, N_drug):
    dp = degp_ref[...]
    dsrc = dp[0:NB, 0:1] + dp[NB:2 * NB, 0:1]
    on = lax.rsqrt(jnp.clip(dsrc[:N_drug], 1.0, None))
    hs_ref[...] = h_ref[...] * on


def _tc_mlp_body(aggp_ref, degp_ref, W_ref, b_ref, W1_ref, b1_ref, g1_ref,
                 be1_ref, W2_ref, b2_ref, g2_ref, be2_ref, W3_ref, b3_ref,
                 s_ref, *, NB, dim, E):
    f32 = jnp.float32
    agg = aggp_ref[0:NB, :] + aggp_ref[NB:2 * NB, :]
    dp = degp_ref[...]
    ddst = dp[0:NB, 1:2] + dp[NB:2 * NB, 1:2]
    in_norm = lax.rsqrt(jnp.clip(ddst, 1.0, None))
    agg2 = agg * in_norm
    h_di = jnp.dot(agg2, W_ref[0:dim, :], preferred_element_type=f32) + b_ref[...]
    z1 = (jnp.dot(h_di, W1_ref[2 * dim:4 * dim, :], preferred_element_type=f32)
          + jnp.dot(b_ref[...], W1_ref[0:2 * dim, :], preferred_element_type=f32)
          + b1_ref[...])
    w = ddst * (1.0 / E)
    m1 = jnp.sum(z1 * w, axis=0, keepdims=True)
    d1 = z1 - m1
    v1 = jnp.sum(d1 * d1 * w, axis=0, keepdims=True)
    z2 = jnp.maximum(d1 / jnp.sqrt(v1 + 1e-5) * g1_ref[...] + be1_ref[...], 0.0)
    x2 = jnp.dot(z2, W2_ref[...], preferred_element_type=f32) + b2_ref[...]
    m2 = jnp.sum(x2 * w, axis=0, keepdims=True)
    d2 = x2 - m2
    v2 = jnp.sum(d2 * d2 * w, axis=0, keepdims=True)
    z3 = jnp.maximum(d2 / jnp.sqrt(v2 + 1e-5) * g2_ref[...] + be2_ref[...], 0.0)
    sc = jnp.dot(z3, W3_ref[...], preferred_element_type=f32) + b3_ref[...]
    s_ref[...] = jax.nn.sigmoid(sc)


def kernel(h_drug, d_disease, edge_index, W, b, W1, b1, g1, be1, W2, b2, g2,
           be2, W3, b3):
    N_drug, dim = h_drug.shape
    N_dis = d_disease.shape[0]
    E = edge_index.shape[1]
    EP = E // NW
    NCH = EP // K
    NB = ((max(N_drug, N_dis) + 127) // 128) * 128

    src = edge_index[0].astype(jnp.int32)
    dst = edge_index[1].astype(jnp.int32)
    src3 = src.reshape(NW, NCH, K)
    dst3 = dst.reshape(NW, NCH, K)

    onesA = jnp.broadcast_to(jnp.eye(dim, dtype=jnp.float32)[:2, None, :], (2, K, dim))
    zerosD = jnp.zeros((NB, dim), jnp.float32)

    degp = _make_sc_degrees(NW, NCH, K, NB, dim)(src3, dst3, onesA, zerosD)

    hs = pl.pallas_call(
        functools.partial(_tc_scale_body, NB=NB, N_drug=N_drug),
        out_shape=jax.ShapeDtypeStruct((N_drug, dim), jnp.float32),
    )(degp, h_drug)

    aggp = _make_sc_agg(NW, NCH, K, NB, dim, N_dis)(hs, src3, dst3, zerosD)

    s = pl.pallas_call(
        functools.partial(_tc_mlp_body, NB=NB, dim=dim, E=E),
        out_shape=jax.ShapeDtypeStruct((NB, 1), jnp.float32),
    )(aggp, degp, W, b.reshape(1, -1), W1, b1.reshape(1, -1),
      g1.reshape(1, -1), be1.reshape(1, -1), W2, b2.reshape(1, -1),
      g2.reshape(1, -1), be2.reshape(1, -1), W3, b3.reshape(1, -1))

    score3 = _make_sc_gather(NW, NCH, K, NB)(s.reshape(NB), dst3)
    return score3.reshape(E)


# speedup vs baseline: 18.9801x; 1.7109x over previous
import functools

import jax
import jax.numpy as jnp
from jax import lax
from jax.experimental import pallas as pl
from jax.experimental.pallas import tpu as pltpu
from jax.experimental.pallas import tpu_sc as plsc

NC = 2
NS = 16
NW = NC * NS
K = 80


def _mesh():
    return plsc.VectorSubcoreMesh(core_axis_name="c", subcore_axis_name="s")


def _make_sc_degrees(NW_, NCH, K_, NB, w):
    rows_per_tile = NB // NS

    @functools.partial(
        pl.kernel,
        mesh=_mesh(),
        out_type=jax.ShapeDtypeStruct((NC * NB, w), jnp.float32),
        scratch_types=[
            pltpu.VMEM((NCH, K_), jnp.int32),
            pltpu.VMEM((NCH, K_), jnp.int32),
            pltpu.VMEM((K_, w), jnp.float32),
            pltpu.VMEM((K_, w), jnp.float32),
            pltpu.VMEM_SHARED((NB, w), jnp.float32),
        ],
    )
    def sc_degrees(src3, dst3, onesA, zerosW, degp, sidx_v, didx_v,
                   ones10_v, ones01_v, hist):
        cid = lax.axis_index("c")
        sid = lax.axis_index("s")
        wid = cid * NS + sid
        pltpu.sync_copy(src3.at[wid], sidx_v)
        pltpu.sync_copy(dst3.at[wid], didx_v)
        pltpu.sync_copy(onesA.at[0], ones10_v)
        pltpu.sync_copy(onesA.at[1], ones01_v)
        pltpu.sync_copy(zerosW.at[pl.ds(sid * rows_per_tile, rows_per_tile)],
                        hist.at[pl.ds(sid * rows_per_tile, rows_per_tile)])
        plsc.subcore_barrier()

        def body(j, carry):
            pltpu.sync_copy(ones10_v, hist.at[sidx_v.at[j]], add=True)
            pltpu.sync_copy(ones01_v, hist.at[didx_v.at[j]], add=True)
            return carry

        lax.fori_loop(0, NCH, body, 0)
        plsc.subcore_barrier()
        pltpu.sync_copy(
            hist.at[pl.ds(sid * rows_per_tile, rows_per_tile)],
            degp.at[pl.ds(cid * NB + sid * rows_per_tile, rows_per_tile)])

    return sc_degrees


def _make_sc_agg(NW_, NCH, K_, NB, dim, n_rows):
    rows_per_tile = NB // NS

    @functools.partial(
        pl.kernel,
        mesh=_mesh(),
        out_type=jax.ShapeDtypeStruct((NC * NB, dim), jnp.float32),
        scratch_types=[
            pltpu.VMEM((NCH, K_), jnp.int32),
            pltpu.VMEM((NCH, K_), jnp.int32),
            pltpu.VMEM((K_, dim), jnp.float32),
            pltpu.VMEM((K_, dim), jnp.float32),
            pltpu.VMEM((K_, dim), jnp.float32),
            pltpu.VMEM((K_, dim), jnp.float32),
            pltpu.VMEM_SHARED((NB, dim), jnp.float32),
            pltpu.SemaphoreType.DMA,
            pltpu.SemaphoreType.DMA,
            pltpu.SemaphoreType.DMA,
            pltpu.SemaphoreType.DMA,
        ],
    )
    def sc_agg(hs, src3, dst3, zerosD, aggp, sidx_v, didx_v, rowsA, rowsB,
               rowsC, rowsD, agg, semA, semB, semC, semD):
        cid = lax.axis_index("c")
        sid = lax.axis_index("s")
        wid = cid * NS + sid
        pltpu.sync_copy(src3.at[wid], sidx_v)
        pltpu.sync_copy(dst3.at[wid], didx_v)
        pltpu.sync_copy(zerosD.at[pl.ds(sid * rows_per_tile, rows_per_tile)],
                        agg.at[pl.ds(sid * rows_per_tile, rows_per_tile)])
        plsc.subcore_barrier()

        rows = (rowsA, rowsB, rowsC, rowsD)
        sems = (semA, semB, semC, semD)
        nbuf = 4
        for b in range(nbuf):
            pltpu.async_copy(hs.at[sidx_v.at[b]], rows[b], sems[b])

        @pl.loop(0, NCH, step=nbuf)
        def grp(g):
            for b in range(nbuf):
                j = g + b

                @pl.when(j < NCH)
                def _():
                    pltpu.make_async_copy(hs.at[pl.ds(0, K_)], rows[b],
                                          sems[b]).wait()
                    pltpu.sync_copy(rows[b], agg.at[didx_v.at[j]], add=True)

                    @pl.when(j + nbuf < NCH)
                    def _():
                        pltpu.async_copy(hs.at[sidx_v.at[j + nbuf]], rows[b],
                                         sems[b])

        plsc.subcore_barrier()
        pltpu.sync_copy(
            agg.at[pl.ds(sid * rows_per_tile, rows_per_tile)],
            aggp.at[pl.ds(cid * NB + sid * rows_per_tile, rows_per_tile)])

    return sc_agg


def _make_sc_gather(NW_, NCH, K_, NB):
    rows_per_tile = NB // NS

    @functools.partial(
        pl.kernel,
        mesh=_mesh(),
        out_type=jax.ShapeDtypeStruct((NW, NCH, K_), jnp.float32),
        scratch_types=[
            pltpu.VMEM((NCH, K_), jnp.int32),
            pltpu.VMEM((NCH, K_), jnp.float32),
            pltpu.VMEM_SHARED((NB,), jnp.float32),
            pltpu.SemaphoreType.DMA,
        ],
    )
    def sc_gather(s1d, dst3, score3, didx_v, out_v, s_sp, sem):
        cid = lax.axis_index("c")
        sid = lax.axis_index("s")
        wid = cid * NS + sid
        pltpu.sync_copy(dst3.at[wid], didx_v)

        @pl.when(sid == 0)
        def _():
            pltpu.sync_copy(s1d, s_sp)

        plsc.subcore_barrier()

        @pl.loop(0, NCH, step=8)
        def grp(g):
            for b in range(8):
                j = g + b

                @pl.when(j < NCH)
                def _():
                    pltpu.async_copy(s_sp.at[didx_v.at[j]], out_v.at[j], sem)

            for b in range(8):
                j = g + b

                @pl.when(j < NCH)
                def _():
                    pltpu.make_async_copy(s1d.at[pl.ds(0, K_)], out_v.at[j],
                                          sem).wait()

        pltpu.sync_copy(out_v, score3.at[wid])

    return sc_gather


def _tc_scale_body(degp_ref, h_ref, hs_ref, *, NB, N_drug):
    dp = degp_ref[...]
    dsrc = dp[0:NB, 0:1] + dp[NB:2 * NB, 0:1]
    on = lax.rsqrt(jnp.clip(dsrc, 1.0, None))
    hs_ref[...] = h_ref[...] * on


def _tc_mlp_body(aggp_ref, degp_ref, W_ref, b_ref, W1_ref, b1_ref, g1_ref,
                 be1_ref, W2_ref, b2_ref, g2_ref, be2_ref, W3_ref, b3_ref,
                 s_ref, *, NB, dim, E):
    f32 = jnp.float32
    agg = aggp_ref[0:NB, :] + aggp_ref[NB:2 * NB, :]
    dp = degp_ref[...]
    ddst = dp[0:NB, 1:2] + dp[NB:2 * NB, 1:2]
    in_norm = lax.rsqrt(jnp.clip(ddst, 1.0, None))
    agg2 = agg * in_norm
    h_di = jnp.dot(agg2, W_ref[0:dim, :], preferred_element_type=f32) + b_ref[...]
    z1 = (jnp.dot(h_di, W1_ref[2 * dim:4 * dim, :], preferred_element_type=f32)
          + jnp.dot(b_ref[...], W1_ref[0:2 * dim, :], preferred_element_type=f32)
          + b1_ref[...])
    w = ddst * (1.0 / E)
    m1 = jnp.sum(z1 * w, axis=0, keepdims=True)
    d1 = z1 - m1
    v1 = jnp.sum(d1 * d1 * w, axis=0, keepdims=True)
    z2 = jnp.maximum(d1 / jnp.sqrt(v1 + 1e-5) * g1_ref[...] + be1_ref[...], 0.0)
    x2 = jnp.dot(z2, W2_ref[...], preferred_element_type=f32) + b2_ref[...]
    m2 = jnp.sum(x2 * w, axis=0, keepdims=True)
    d2 = x2 - m2
    v2 = jnp.sum(d2 * d2 * w, axis=0, keepdims=True)
    z3 = jnp.maximum(d2 / jnp.sqrt(v2 + 1e-5) * g2_ref[...] + be2_ref[...], 0.0)
    sc = jnp.dot(z3, W3_ref[...], preferred_element_type=f32) + b3_ref[...]
    s_ref[...] = jax.nn.sigmoid(sc)


def kernel(h_drug, d_disease, edge_index, W, b, W1, b1, g1, be1, W2, b2, g2,
           be2, W3, b3):
    N_drug, dim = h_drug.shape
    N_dis = d_disease.shape[0]
    E = edge_index.shape[1]
    EP = E // NW
    NCH = EP // K
    NB = ((max(N_drug, N_dis) + 127) // 128) * 128

    src = edge_index[0].astype(jnp.int32)
    dst = edge_index[1].astype(jnp.int32)
    src3 = src.reshape(NW, NCH, K)
    dst3 = dst.reshape(NW, NCH, K)

    onesA = jnp.broadcast_to(jnp.eye(dim, dtype=jnp.float32)[:2, None, :], (2, K, dim))
    zerosD = jnp.zeros((NB, dim), jnp.float32)

    degp = _make_sc_degrees(NW, NCH, K, NB, dim)(src3, dst3, onesA, zerosD)

    h_pad = jnp.zeros((NB, dim), jnp.float32).at[:N_drug].set(h_drug)
    hs = pl.pallas_call(
        functools.partial(_tc_scale_body, NB=NB, N_drug=N_drug),
        out_shape=jax.ShapeDtypeStruct((NB, dim), jnp.float32),
    )(degp, h_pad)

    aggp = _make_sc_agg(NW, NCH, K, NB, dim, N_dis)(hs, src3, dst3, zerosD)

    s = pl.pallas_call(
        functools.partial(_tc_mlp_body, NB=NB, dim=dim, E=E),
        out_shape=jax.ShapeDtypeStruct((NB, 1), jnp.float32),
    )(aggp, degp, W, b.reshape(1, -1), W1, b1.reshape(1, -1),
      g1.reshape(1, -1), be1.reshape(1, -1), W2, b2.reshape(1, -1),
      g2.reshape(1, -1), be2.reshape(1, -1), W3, b3.reshape(1, -1))

    score3 = _make_sc_gather(NW, NCH, K, NB)(s.reshape(NB), dst3)
    return score3.reshape(E)


# speedup vs baseline: 30.4191x; 1.6027x over previous
import functools

import jax
import jax.numpy as jnp
from jax import lax
from jax.experimental import pallas as pl
from jax.experimental.pallas import tpu as pltpu
from jax.experimental.pallas import tpu_sc as plsc

NC = 2
NS = 16
NW = NC * NS
K = 80


def _mesh():
    return plsc.VectorSubcoreMesh(core_axis_name="c", subcore_axis_name="s")


def _make_sc_degrees(NW_, NCH, K_, NB):
    @functools.partial(
        pl.kernel,
        mesh=_mesh(),
        out_type=jax.ShapeDtypeStruct((2, NW, NB // 128, 128), jnp.float32),
        compiler_params=pltpu.CompilerParams(needs_layout_passes=False),
        scratch_types=[
            pltpu.VMEM((NCH, K_), jnp.int32),
            pltpu.VMEM((NCH, K_), jnp.int32),
            pltpu.VMEM((NB // 128, 128), jnp.float32),
            pltpu.VMEM((NB // 128, 128), jnp.float32),
        ],
    )
    def sc_degrees(src3, dst3, zeros2, degp, sidx_v, didx_v, hs_v, hd_v):
        cid = lax.axis_index("c")
        sid = lax.axis_index("s")
        wid = cid * NS + sid
        pltpu.sync_copy(src3.at[wid], sidx_v)
        pltpu.sync_copy(dst3.at[wid], didx_v)
        pltpu.sync_copy(zeros2, hs_v)
        pltpu.sync_copy(zeros2, hd_v)

        ones16 = jnp.ones((16,), jnp.float32)

        @pl.loop(0, NCH)
        def chunk(j):
            for b in range(K_ // 16):
                s16 = sidx_v[j, pl.ds(b * 16, 16)]
                d16 = didx_v[j, pl.ds(b * 16, 16)]
                plsc.addupdate_scatter(hs_v, [s16 >> 7, s16 & 127], ones16)
                plsc.addupdate_scatter(hd_v, [d16 >> 7, d16 & 127], ones16)

        pltpu.sync_copy(hs_v, degp.at[0, wid])
        pltpu.sync_copy(hd_v, degp.at[1, wid])

    return sc_degrees


def _make_sc_agg(NW_, NCH, K_, NB, dim, n_rows):
    rows_per_tile = NB // NS

    @functools.partial(
        pl.kernel,
        mesh=_mesh(),
        out_type=jax.ShapeDtypeStruct((NC * NB, dim), jnp.float32),
        scratch_types=[
            pltpu.VMEM((NCH, K_), jnp.int32),
            pltpu.VMEM((NCH, K_), jnp.int32),
            pltpu.VMEM((K_, dim), jnp.float32),
            pltpu.VMEM((K_, dim), jnp.float32),
            pltpu.VMEM((K_, dim), jnp.float32),
            pltpu.VMEM((K_, dim), jnp.float32),
            pltpu.VMEM_SHARED((NB, dim), jnp.float32),
            pltpu.SemaphoreType.DMA,
            pltpu.SemaphoreType.DMA,
            pltpu.SemaphoreType.DMA,
            pltpu.SemaphoreType.DMA,
        ],
    )
    def sc_agg(hs, src3, dst3, zerosD, aggp, sidx_v, didx_v, rowsA, rowsB,
               rowsC, rowsD, agg, semA, semB, semC, semD):
        cid = lax.axis_index("c")
        sid = lax.axis_index("s")
        wid = cid * NS + sid
        pltpu.sync_copy(src3.at[wid], sidx_v)
        pltpu.sync_copy(dst3.at[wid], didx_v)
        pltpu.sync_copy(zerosD.at[pl.ds(sid * rows_per_tile, rows_per_tile)],
                        agg.at[pl.ds(sid * rows_per_tile, rows_per_tile)])
        plsc.subcore_barrier()

        rows = (rowsA, rowsB, rowsC, rowsD)
        sems = (semA, semB, semC, semD)
        nbuf = 4
        for b in range(nbuf):
            pltpu.async_copy(hs.at[sidx_v.at[b]], rows[b], sems[b])

        @pl.loop(0, NCH, step=nbuf)
        def grp(g):
            for b in range(nbuf):
                j = g + b

                @pl.when(j < NCH)
                def _():
                    pltpu.make_async_copy(hs.at[pl.ds(0, K_)], rows[b],
                                          sems[b]).wait()
                    pltpu.sync_copy(rows[b], agg.at[didx_v.at[j]], add=True)

                    @pl.when(j + nbuf < NCH)
                    def _():
                        pltpu.async_copy(hs.at[sidx_v.at[j + nbuf]], rows[b],
                                         sems[b])

        plsc.subcore_barrier()
        pltpu.sync_copy(
            agg.at[pl.ds(sid * rows_per_tile, rows_per_tile)],
            aggp.at[pl.ds(cid * NB + sid * rows_per_tile, rows_per_tile)])

    return sc_agg


def _make_sc_gather(NW_, NCH, K_, NB):
    rows_per_tile = NB // NS

    @functools.partial(
        pl.kernel,
        mesh=_mesh(),
        out_type=jax.ShapeDtypeStruct((NW, NCH, K_), jnp.float32),
        scratch_types=[
            pltpu.VMEM((NCH, K_), jnp.int32),
            pltpu.VMEM((NCH, K_), jnp.float32),
            pltpu.VMEM_SHARED((NB,), jnp.float32),
            pltpu.SemaphoreType.DMA,
        ],
    )
    def sc_gather(s1d, dst3, score3, didx_v, out_v, s_sp, sem):
        cid = lax.axis_index("c")
        sid = lax.axis_index("s")
        wid = cid * NS + sid
        pltpu.sync_copy(dst3.at[wid], didx_v)

        @pl.when(sid == 0)
        def _():
            pltpu.sync_copy(s1d, s_sp)

        plsc.subcore_barrier()

        @pl.loop(0, NCH, step=8)
        def grp(g):
            for b in range(8):
                j = g + b

                @pl.when(j < NCH)
                def _():
                    pltpu.async_copy(s_sp.at[didx_v.at[j]], out_v.at[j], sem)

            for b in range(8):
                j = g + b

                @pl.when(j < NCH)
                def _():
                    pltpu.make_async_copy(s1d.at[pl.ds(0, K_)], out_v.at[j],
                                          sem).wait()

        pltpu.sync_copy(out_v, score3.at[wid])

    return sc_gather


def _tc_scale_body(degp_ref, h_ref, hs_ref, *, NB, N_drug):
    dsrc = jnp.sum(degp_ref[0], axis=1, keepdims=True)
    on = lax.rsqrt(jnp.clip(dsrc, 1.0, None))
    hs_ref[...] = h_ref[...] * on


def _tc_mlp_body(aggp_ref, degp_ref, W_ref, b_ref, W1_ref, b1_ref, g1_ref,
                 be1_ref, W2_ref, b2_ref, g2_ref, be2_ref, W3_ref, b3_ref,
                 s_ref, *, NB, dim, E):
    f32 = jnp.float32
    agg = aggp_ref[0:NB, :] + aggp_ref[NB:2 * NB, :]
    ddst = jnp.sum(degp_ref[1], axis=1, keepdims=True)
    in_norm = lax.rsqrt(jnp.clip(ddst, 1.0, None))
    agg2 = agg * in_norm
    h_di = jnp.dot(agg2, W_ref[0:dim, :], preferred_element_type=f32) + b_ref[...]
    z1 = (jnp.dot(h_di, W1_ref[2 * dim:4 * dim, :], preferred_element_type=f32)
          + jnp.dot(b_ref[...], W1_ref[0:2 * dim, :], preferred_element_type=f32)
          + b1_ref[...])
    w = ddst * (1.0 / E)
    m1 = jnp.sum(z1 * w, axis=0, keepdims=True)
    d1 = z1 - m1
    v1 = jnp.sum(d1 * d1 * w, axis=0, keepdims=True)
    z2 = jnp.maximum(d1 / jnp.sqrt(v1 + 1e-5) * g1_ref[...] + be1_ref[...], 0.0)
    x2 = jnp.dot(z2, W2_ref[...], preferred_element_type=f32) + b2_ref[...]
    m2 = jnp.sum(x2 * w, axis=0, keepdims=True)
    d2 = x2 - m2
    v2 = jnp.sum(d2 * d2 * w, axis=0, keepdims=True)
    z3 = jnp.maximum(d2 / jnp.sqrt(v2 + 1e-5) * g2_ref[...] + be2_ref[...], 0.0)
    sc = jnp.dot(z3, W3_ref[...], preferred_element_type=f32) + b3_ref[...]
    s_ref[...] = jax.nn.sigmoid(sc)


def kernel(h_drug, d_disease, edge_index, W, b, W1, b1, g1, be1, W2, b2, g2,
           be2, W3, b3):
    N_drug, dim = h_drug.shape
    N_dis = d_disease.shape[0]
    E = edge_index.shape[1]
    EP = E // NW
    NCH = EP // K
    NB = ((max(N_drug, N_dis) + 127) // 128) * 128

    src = edge_index[0].astype(jnp.int32)
    dst = edge_index[1].astype(jnp.int32)
    src3 = src.reshape(NW, NCH, K)
    dst3 = dst.reshape(NW, NCH, K)

    zerosD = jnp.zeros((NB, dim), jnp.float32)
    zeros2 = jnp.zeros((NB // 128, 128), jnp.float32)

    degp = _make_sc_degrees(NW, NCH, K, NB)(src3, dst3, zeros2)
    degp = degp.reshape(2, NW, NB).transpose(0, 2, 1)

    h_pad = jnp.zeros((NB, dim), jnp.float32).at[:N_drug].set(h_drug)
    hs = pl.pallas_call(
        functools.partial(_tc_scale_body, NB=NB, N_drug=N_drug),
        out_shape=jax.ShapeDtypeStruct((NB, dim), jnp.float32),
    )(degp, h_pad)

    aggp = _make_sc_agg(NW, NCH, K, NB, dim, N_dis)(hs, src3, dst3, zerosD)

    s = pl.pallas_call(
        functools.partial(_tc_mlp_body, NB=NB, dim=dim, E=E),
        out_shape=jax.ShapeDtypeStruct((NB, 1), jnp.float32),
    )(aggp, degp, W, b.reshape(1, -1), W1, b1.reshape(1, -1),
      g1.reshape(1, -1), be1.reshape(1, -1), W2, b2.reshape(1, -1),
      g2.reshape(1, -1), be2.reshape(1, -1), W3, b3.reshape(1, -1))

    score3 = _make_sc_gather(NW, NCH, K, NB)(s.reshape(NB), dst3)
    return score3.reshape(E)
